# baseline (speedup 1.0000x reference)
import functools

import jax
import jax.numpy as jnp
from jax import lax
from jax.experimental import pallas as pl
from jax.experimental.pallas import tpu as pltpu
from jax.experimental.pallas import tpu_sc as plsc

N = 10000
E = 320000
D = 128

NC = 2
NS = 16
NW = NC * NS
LANES = 16

CHUNK = 40
EW = E // NW
NCHUNK = EW // CHUNK
RPT = 624
TAIL0 = RPT * NS
TAILN = N - TAIL0


@functools.cache
def _make_sc_gather34():
    mesh = plsc.VectorSubcoreMesh(core_axis_name="c", subcore_axis_name="s")

    @functools.partial(
        pl.kernel,
        out_type=(
            jax.ShapeDtypeStruct((E, D), jnp.float32),
            jax.ShapeDtypeStruct((NC, N, D), jnp.float32),
        ),
        mesh=mesh,
        scratch_types=[
            pltpu.VMEM((CHUNK,), jnp.int32),
            pltpu.VMEM((CHUNK,), jnp.int32),
            pltpu.VMEM((CHUNK,), jnp.int32),
            pltpu.VMEM((CHUNK,), jnp.int32),
            pltpu.VMEM((CHUNK, D), jnp.float32),
            pltpu.VMEM((CHUNK, D), jnp.float32),
            pltpu.VMEM((CHUNK, D), jnp.float32),
            pltpu.SemaphoreType.DMA,
            pltpu.SemaphoreType.DMA,
            pltpu.SemaphoreType.DMA,
            pltpu.SemaphoreType.DMA,
            pltpu.SemaphoreType.DMA,
            pltpu.SemaphoreType.DMA,
            pltpu.SemaphoreType.DMA,
            pltpu.SemaphoreType.DMA,
            pltpu.SemaphoreType.DMA,
            pltpu.SemaphoreType.DMA,
            pltpu.SemaphoreType.DMA,
            pltpu.SemaphoreType.DMA,
            pltpu.VMEM_SHARED((N, D), jnp.float32),
        ],
    )
    def sc_gather34(x3_hbm, src_hbm, g3_hbm, cnt_hbm,
                    is0, is1, is2, is3,
                    r3a, r3b, ones_v,
                    si0, si1, si2, si3, s3a, s3b, w3a, w3b,
                    sc0, sc1, sc2, sc3, cnt_sh):
        cid = lax.axis_index("c")
        sid = lax.axis_index("s")
        wid = sid * NC + cid
        base = wid * EW
        isl = (is0, is1, is2, is3)
        si = (si0, si1, si2, si3)
        r3 = (r3a, r3b)
        s3 = (s3a, s3b)
        w3 = (w3a, w3b)
        sc = (sc0, sc1, sc2, sc3)

        @pl.loop(0, CHUNK)
        def _(i):
            for j in range(D // LANES):
                ones_v[i, pl.ds(j * LANES, LANES)] = jnp.ones((LANES,), jnp.float32)
                r3a[i, pl.ds(j * LANES, LANES)] = jnp.zeros((LANES,), jnp.float32)

        zr0 = sid * RPT
        for k in range(RPT // CHUNK):
            pltpu.sync_copy(r3a, cnt_sh.at[pl.ds(zr0 + k * CHUNK, CHUNK)])
        _t0 = (RPT // CHUNK) * CHUNK
        _tn = RPT - _t0
        pltpu.sync_copy(r3a.at[pl.ds(0, _tn)], cnt_sh.at[pl.ds(zr0 + _t0, _tn)])

        @pl.when(sid == NS - 1)
        def _():
            pltpu.sync_copy(r3a.at[pl.ds(0, TAILN)],
                            cnt_sh.at[pl.ds(TAIL0, TAILN)])

        plsc.subcore_barrier()

        def idx_load(c, s):
            eb = base + c * CHUNK
            pltpu.async_copy(src_hbm.at[pl.ds(eb, CHUNK)], isl[s], si[s])

        def idx_wait(c, s):
            eb = base + c * CHUNK
            pltpu.make_async_copy(src_hbm.at[pl.ds(eb, CHUNK)], isl[s], si[s]).wait()

        def fetch(c, b, s):
            pltpu.async_copy(x3_hbm.at[isl[s]], r3[b], s3[b])

        def cnt_issue(s):
            pltpu.async_copy(ones_v, cnt_sh.at[isl[s]], sc[s], add=True)

        def cnt_wait(s):
            pltpu.make_async_copy(ones_v, cnt_sh.at[isl[s]], sc[s]).wait()

        def process(c, b, s, pf_i, pf_g):
            eb = base + c * CHUNK
            pltpu.make_async_copy(x3_hbm.at[isl[s]], r3[b], s3[b]).wait()
            pltpu.async_copy(r3[b], g3_hbm.at[pl.ds(eb, CHUNK)], w3[b])
            if pf_i:
                cnt_wait(s)
                idx_load(c + 4, s)
            if pf_g:
                s2 = (s + 2) % 4
                idx_wait(c + 2, s2)
                cnt_issue(s2)
                pltpu.make_async_copy(r3[b], g3_hbm.at[pl.ds(eb, CHUNK)], w3[b]).wait()
                fetch(c + 2, b, s2)
            else:
                pltpu.make_async_copy(r3[b], g3_hbm.at[pl.ds(eb, CHUNK)], w3[b]).wait()

        idx_load(0, 0)
        idx_load(1, 1)
        idx_load(2, 2)
        idx_load(3, 3)
        idx_wait(0, 0)
        cnt_issue(0)
        fetch(0, 0, 0)
        idx_wait(1, 1)
        cnt_issue(1)
        fetch(1, 1, 1)

        @pl.loop(0, NCHUNK - 6, step=4)
        def _(ci):
            process(ci + 0, 0, 0, True, True)
            process(ci + 1, 1, 1, True, True)
            process(ci + 2, 0, 2, True, True)
            process(ci + 3, 1, 3, True, True)

        process(NCHUNK - 6, 0, 0, True, True)
        process(NCHUNK - 5, 1, 1, True, True)
        process(NCHUNK - 4, 0, 2, False, True)
        process(NCHUNK - 3, 1, 3, False, True)
        process(NCHUNK - 2, 0, 0, False, False)
        process(NCHUNK - 1, 1, 1, False, False)

        cnt_wait(2)
        cnt_wait(3)
        cnt_wait(0)
        cnt_wait(1)

        plsc.subcore_barrier()

        r0 = sid * RPT
        pltpu.sync_copy(cnt_sh.at[pl.ds(r0, RPT)],
                        cnt_hbm.at[cid, pl.ds(r0, RPT)])

        @pl.when(sid == NS - 1)
        def _():
            pltpu.sync_copy(cnt_sh.at[pl.ds(TAIL0, TAILN)],
                            cnt_hbm.at[cid, pl.ds(TAIL0, TAILN)])

    return sc_gather34


@functools.cache
def _make_sc_msg_scatter():
    mesh = plsc.VectorSubcoreMesh(core_axis_name="c", subcore_axis_name="s")

    @functools.partial(
        pl.kernel,
        out_type=(
            jax.ShapeDtypeStruct((NC, N, D), jnp.float32),
            jax.ShapeDtypeStruct((E, D), jnp.float32),
        ),
        mesh=mesh,
        scratch_types=[
            pltpu.VMEM((CHUNK,), jnp.int32),
            pltpu.VMEM((CHUNK,), jnp.int32),
            pltpu.VMEM((CHUNK,), jnp.int32),
            pltpu.VMEM((CHUNK,), jnp.int32),
            pltpu.VMEM((CHUNK,), jnp.int32),
            pltpu.VMEM((CHUNK,), jnp.int32),
            pltpu.VMEM((CHUNK,), jnp.int32),
            pltpu.VMEM((CHUNK,), jnp.int32),
            pltpu.VMEM((CHUNK, D), jnp.float32),
            pltpu.VMEM((CHUNK, D), jnp.float32),
            pltpu.VMEM((CHUNK, D), jnp.float32),
            pltpu.VMEM((CHUNK, D), jnp.float32),
            pltpu.VMEM((CHUNK, D), jnp.float32),
            pltpu.VMEM((CHUNK, D), jnp.float32),
            pltpu.VMEM_SHARED((N, D), jnp.float32),
            pltpu.SemaphoreType.DMA,
            pltpu.SemaphoreType.DMA,
            pltpu.SemaphoreType.DMA,
            pltpu.SemaphoreType.DMA,
            pltpu.SemaphoreType.DMA,
            pltpu.SemaphoreType.DMA,
            pltpu.SemaphoreType.DMA,
            pltpu.SemaphoreType.DMA,
            pltpu.SemaphoreType.DMA,
            pltpu.SemaphoreType.DMA,
            pltpu.SemaphoreType.DMA,
            pltpu.SemaphoreType.DMA,
        ],
    )
    def sc_msg_scatter(x2_hbm, x4_hbm, sig_hbm, src_hbm, dst_hbm,
                       sum_hbm, g4_hbm,
                       is0, is1, is2, is3, id0, id1, id2, id3,
                       rows0, rows1, sig0, sig1, r4a, r4b, acc_sh,
                       si0, si1, si2, si3, sg0, sg1, ss0, ss1,
                       s4a, s4b, w4a, w4b):
        cid = lax.axis_index("c")
        sid = lax.axis_index("s")
        wid = sid * NC + cid
        base = wid * EW
        isl = (is0, is1, is2, is3)
        idl = (id0, id1, id2, id3)
        si = (si0, si1, si2, si3)
        rows = (rows0, rows1)
        sigv = (sig0, sig1)
        r4 = (r4a, r4b)
        sg = (sg0, sg1)
        ss = (ss0, ss1)
        s4 = (s4a, s4b)
        w4 = (w4a, w4b)

        @pl.loop(0, CHUNK)
        def _(i):
            for j in range(D // LANES):
                rows0[i, pl.ds(j * LANES, LANES)] = jnp.zeros((LANES,), jnp.float32)

        zr0 = sid * RPT
        for k in range(RPT // CHUNK):
            pltpu.sync_copy(rows0, acc_sh.at[pl.ds(zr0 + k * CHUNK, CHUNK)])
        _t0 = (RPT // CHUNK) * CHUNK
        _tn = RPT - _t0
        pltpu.sync_copy(rows0.at[pl.ds(0, _tn)], acc_sh.at[pl.ds(zr0 + _t0, _tn)])

        @pl.when(sid == NS - 1)
        def _():
            pltpu.sync_copy(rows0.at[pl.ds(0, TAILN)],
                            acc_sh.at[pl.ds(TAIL0, TAILN)])

        plsc.subcore_barrier()

        def idx_load(c, s):
            eb = base + c * CHUNK
            pltpu.async_copy(src_hbm.at[pl.ds(eb, CHUNK)], isl[s], si[s])
            pltpu.async_copy(dst_hbm.at[pl.ds(eb, CHUNK)], idl[s], si[s])

        def idx_wait(c, s):
            eb = base + c * CHUNK
            pltpu.make_async_copy(src_hbm.at[pl.ds(eb, CHUNK)], isl[s], si[s]).wait()
            pltpu.make_async_copy(dst_hbm.at[pl.ds(eb, CHUNK)], idl[s], si[s]).wait()

        def fetch(c, b, s):
            eb = base + c * CHUNK
            pltpu.async_copy(x2_hbm.at[idl[s]], rows[b], sg[b])
            pltpu.async_copy(x4_hbm.at[idl[s]], r4[b], s4[b])
            pltpu.async_copy(sig_hbm.at[pl.ds(eb, CHUNK)], sigv[b], ss[b])

        def process(c, b, s, pf_i, pf_g):
            eb = base + c * CHUNK
            pltpu.make_async_copy(x4_hbm.at[idl[s]], r4[b], s4[b]).wait()
            pltpu.async_copy(r4[b], g4_hbm.at[pl.ds(eb, CHUNK)], w4[b])
            pltpu.make_async_copy(x2_hbm.at[idl[s]], rows[b], sg[b]).wait()
            pltpu.make_async_copy(sig_hbm.at[pl.ds(eb, CHUNK)], sigv[b], ss[b]).wait()

            @pl.loop(0, CHUNK, step=2)
            def _(i):
                for u in range(2):
                    for j in range(D // LANES):
                        sl = (i + u, pl.ds(j * LANES, LANES))
                        rows[b][sl] = rows[b][sl] * sigv[b][sl]

            pltpu.sync_copy(rows[b], acc_sh.at[isl[s]], add=True)
            if pf_i:
                idx_load(c + 4, s)
            if pf_g:
                s2 = (s + 2) % 4
                idx_wait(c + 2, s2)
                pltpu.make_async_copy(r4[b], g4_hbm.at[pl.ds(eb, CHUNK)], w4[b]).wait()
                fetch(c + 2, b, s2)
            else:
                pltpu.make_async_copy(r4[b], g4_hbm.at[pl.ds(eb, CHUNK)], w4[b]).wait()

        idx_load(0, 0)
        idx_load(1, 1)
        idx_load(2, 2)
        idx_load(3, 3)
        idx_wait(0, 0)
        fetch(0, 0, 0)
        idx_wait(1, 1)
        fetch(1, 1, 1)

        @pl.loop(0, NCHUNK - 6, step=4)
        def _(ci):
            process(ci + 0, 0, 0, True, True)
            process(ci + 1, 1, 1, True, True)
            process(ci + 2, 0, 2, True, True)
            process(ci + 3, 1, 3, True, True)

        process(NCHUNK - 6, 0, 0, True, True)
        process(NCHUNK - 5, 1, 1, True, True)
        process(NCHUNK - 4, 0, 2, False, True)
        process(NCHUNK - 3, 1, 3, False, True)
        process(NCHUNK - 2, 0, 0, False, False)
        process(NCHUNK - 1, 1, 1, False, False)

        plsc.subcore_barrier()

        r0 = sid * RPT
        pltpu.sync_copy(acc_sh.at[pl.ds(r0, RPT)],
                        sum_hbm.at[cid, pl.ds(r0, RPT)])

        @pl.when(sid == NS - 1)
        def _():
            pltpu.sync_copy(acc_sh.at[pl.ds(TAIL0, TAILN)],
                            sum_hbm.at[cid, pl.ds(TAIL0, TAILN)])

    return sc_msg_scatter


_NB = 1000
_EB = 4000


def _node_mm_body(x_ref, w1_ref, b1_ref, w2_ref, b2_ref, w3_ref, b3_ref,
                  w4_ref, b4_ref, o1_ref, o2_ref, o3_ref, o4_ref):
    xb = x_ref[...]
    o1_ref[...] = jnp.dot(xb, w1_ref[...], preferred_element_type=jnp.float32) + b1_ref[...]
    o2_ref[...] = jnp.dot(xb, w2_ref[...], preferred_element_type=jnp.float32) + b2_ref[...]
    o3_ref[...] = jnp.dot(xb, w3_ref[...], preferred_element_type=jnp.float32) + b3_ref[...]
    o4_ref[...] = jnp.dot(xb, w4_ref[...], preferred_element_type=jnp.float32) + b4_ref[...]


def _node_mm(x, w1t, b1, w2t, b2, w3t, b3, w4t, b4):
    row = pl.BlockSpec((_NB, D), lambda i: (i, 0))
    full = pl.BlockSpec((D, D), lambda i: (0, 0))
    bias = pl.BlockSpec((1, D), lambda i: (0, 0))
    o = jax.ShapeDtypeStruct((N, D), jnp.float32)
    return pl.pallas_call(
        _node_mm_body,
        grid=(N // _NB,),
        in_specs=[row, full, bias, full, bias, full, bias, full, bias],
        out_specs=[row, row, row, row],
        out_shape=[o, o, o, o],
    )(x, w1t, b1, w2t, b2, w3t, b3, w4t, b4)


def _edge1_body(w0_ref, we_ref, bwe_ref, sig_ref, w1_ref):
    w0b = w0_ref[...]
    sig_ref[...] = jax.nn.sigmoid(w0b)
    w1_ref[...] = jnp.dot(w0b, we_ref[...], preferred_element_type=jnp.float32) + bwe_ref[...]


def _edge1(w0, wet, bwe):
    row = pl.BlockSpec((_EB, D), lambda i: (i, 0))
    o = jax.ShapeDtypeStruct((E, D), jnp.float32)
    return pl.pallas_call(
        _edge1_body,
        grid=(E // _EB,),
        in_specs=[row, pl.BlockSpec((D, D), lambda i: (0, 0)),
                  pl.BlockSpec((1, D), lambda i: (0, 0))],
        out_specs=[row, row],
        out_shape=[o, o],
    )(w0, wet, bwe)


def _edge_stats_body(w1_ref, g3_ref, g4_ref, s_ref, q_ref):
    i = pl.program_id(0)
    t = w1_ref[...] + g3_ref[...] + g4_ref[...]
    ts = t.reshape(_EB // 8, 8, D)
    ps = jnp.sum(ts, axis=0)
    pq = jnp.sum(ts * ts, axis=0)

    @pl.when(i == 0)
    def _():
        s_ref[...] = jnp.zeros_like(s_ref)
        q_ref[...] = jnp.zeros_like(q_ref)

    s_ref[...] += ps
    q_ref[...] += pq


def _edge_stats(w1, g3, g4):
    row = pl.BlockSpec((_EB, D), lambda i: (i, 0))
    acc = pl.BlockSpec((8, D), lambda i: (0, 0))
    o = jax.ShapeDtypeStruct((8, D), jnp.float32)
    return pl.pallas_call(
        _edge_stats_body,
        grid=(E // _EB,),
        in_specs=[row, row, row],
        out_specs=[acc, acc],
        out_shape=[o, o],
    )(w1, g3, g4)


def _edge_fin_body(w0_ref, w1_ref, g3_ref, g4_ref, s_ref, q_ref,
                   ge_ref, be_ref, o_ref):
    s = jnp.sum(s_ref[...], axis=0, keepdims=True)
    q = jnp.sum(q_ref[...], axis=0, keepdims=True)
    m = s / E
    v = q / E - m * m
    r = lax.rsqrt(v + 1e-5)
    t = w1_ref[...] + g3_ref[...] + g4_ref[...]
    h = (t - m) * r * ge_ref[...] + be_ref[...]
    o_ref[...] = w0_ref[...] + h * jax.nn.sigmoid(h)


def _edge_fin(w0, w1, g3, g4, ssum, ssq, ge, be):
    row = pl.BlockSpec((_EB, D), lambda i: (i, 0))
    acc = pl.BlockSpec((8, D), lambda i: (0, 0))
    bias = pl.BlockSpec((1, D), lambda i: (0, 0))
    return pl.pallas_call(
        _edge_fin_body,
        grid=(E // _EB,),
        in_specs=[row, row, row, row, acc, acc, bias, bias],
        out_specs=row,
        out_shape=jax.ShapeDtypeStruct((E, D), jnp.float32),
    )(w0, w1, g3, g4, ssum, ssq, ge, be)


def _node_fin_body(x_ref, x1_ref, sp_ref, cp_ref, gv_ref, bv_ref, o_ref):
    s = sp_ref[0] + sp_ref[1]
    c = cp_ref[0, :, 0:1] + cp_ref[1, :, 0:1]
    agg = s / jnp.maximum(c, 1.0)
    h = x1_ref[...] + agg
    m = jnp.mean(h, axis=0, keepdims=True)
    v = jnp.mean((h - m) ** 2, axis=0, keepdims=True)
    hn = (h - m) * lax.rsqrt(v + 1e-5) * gv_ref[...] + bv_ref[...]
    o_ref[...] = x_ref[...] + hn * jax.nn.sigmoid(hn)


def _node_fin(x, x1, sum_p, cnt_p, gv, bv):
    return pl.pallas_call(
        _node_fin_body,
        grid=(1,),
        in_specs=[
            pl.BlockSpec((N, D), lambda i: (0, 0)),
            pl.BlockSpec((N, D), lambda i: (0, 0)),
            pl.BlockSpec((NC, N, D), lambda i: (0, 0, 0)),
            pl.BlockSpec((NC, N, D), lambda i: (0, 0, 0)),
            pl.BlockSpec((1, D), lambda i: (0, 0)),
            pl.BlockSpec((1, D), lambda i: (0, 0)),
        ],
        out_specs=pl.BlockSpec((N, D), lambda i: (0, 0)),
        out_shape=jax.ShapeDtypeStruct((N, D), jnp.float32),
    )(x, x1, sum_p, cnt_p, gv, bv)


def kernel(x, edge_index, edge_attr, W1, b1, W2, b2, W3, b3, W4, b4,
           We, bwe, g_v, beta_v, g_e, beta_e):
    src = edge_index[0]
    dst = edge_index[1]

    x1, x2, x3, x4 = _node_mm(
        x, W1.T, b1.reshape(1, D), W2.T, b2.reshape(1, D),
        W3.T, b3.reshape(1, D), W4.T, b4.reshape(1, D))

    sig, w1 = _edge1(edge_attr, We.T, bwe.reshape(1, D))

    g3, cnt_p = _make_sc_gather34()(x3, src)
    sum_p, g4 = _make_sc_msg_scatter()(x2, x4, sig, src, dst)

    ssum, ssq = _edge_stats(w1, g3, g4)

    x_out = _node_fin(x, x1, sum_p, cnt_p,
                      g_v.reshape(1, D), beta_v.reshape(1, D))
    w_out = _edge_fin(edge_attr, w1, g3, g4, ssum, ssq,
                      g_e.reshape(1, D), beta_e.reshape(1, D))
    return (x_out, w_out)


# speedup vs baseline: 1.1774x; 1.1774x over previous
import functools

import jax
import jax.numpy as jnp
from jax import lax
from jax.experimental import pallas as pl
from jax.experimental.pallas import tpu as pltpu
from jax.experimental.pallas import tpu_sc as plsc

N = 10000
E = 320000
D = 128

NC = 2
NS = 16
NW = NC * NS
LANES = 16

CHUNK = 40
EW = E // NW
NCHUNK = EW // CHUNK
RPT = 624
TAIL0 = RPT * NS
TAILN = N - TAIL0


@functools.cache
def _make_sc_gather34():
    mesh = plsc.VectorSubcoreMesh(core_axis_name="c", subcore_axis_name="s")

    @functools.partial(
        pl.kernel,
        out_type=(
            jax.ShapeDtypeStruct((E, D), jnp.float32),
            jax.ShapeDtypeStruct((E, D), jnp.float32),
            jax.ShapeDtypeStruct((NC, N, D), jnp.float32),
        ),
        mesh=mesh,
        scratch_types=[
            pltpu.VMEM((CHUNK,), jnp.int32),
            pltpu.VMEM((CHUNK,), jnp.int32),
            pltpu.VMEM((CHUNK,), jnp.int32),
            pltpu.VMEM((CHUNK,), jnp.int32),
            pltpu.VMEM((CHUNK,), jnp.int32),
            pltpu.VMEM((CHUNK,), jnp.int32),
            pltpu.VMEM((CHUNK,), jnp.int32),
            pltpu.VMEM((CHUNK,), jnp.int32),
            pltpu.VMEM((CHUNK, D), jnp.float32),
            pltpu.VMEM((CHUNK, D), jnp.float32),
            pltpu.VMEM((CHUNK, D), jnp.float32),
            pltpu.VMEM((CHUNK, D), jnp.float32),
            pltpu.VMEM((CHUNK, D), jnp.float32),
            pltpu.SemaphoreType.DMA,
            pltpu.SemaphoreType.DMA,
            pltpu.SemaphoreType.DMA,
            pltpu.SemaphoreType.DMA,
            pltpu.SemaphoreType.DMA,
            pltpu.SemaphoreType.DMA,
            pltpu.SemaphoreType.DMA,
            pltpu.SemaphoreType.DMA,
            pltpu.SemaphoreType.DMA,
            pltpu.SemaphoreType.DMA,
            pltpu.SemaphoreType.DMA,
            pltpu.SemaphoreType.DMA,
            pltpu.SemaphoreType.DMA,
            pltpu.SemaphoreType.DMA,
            pltpu.SemaphoreType.DMA,
            pltpu.SemaphoreType.DMA,
            pltpu.VMEM_SHARED((N, D), jnp.float32),
        ],
    )
    def sc_gather34(x3_hbm, x4_hbm, src_hbm, dst_hbm, g3_hbm, g4_hbm, cnt_hbm,
                    is0, is1, is2, is3, id0, id1, id2, id3,
                    r3a, r3b, r4a, r4b, ones_v,
                    si0, si1, si2, si3, s3a, s3b, s4a, s4b, w3a, w3b, w4a, w4b,
                    sc0, sc1, sc2, sc3, cnt_sh):
        cid = lax.axis_index("c")
        sid = lax.axis_index("s")
        wid = sid * NC + cid
        base = wid * EW
        isl = (is0, is1, is2, is3)
        idl = (id0, id1, id2, id3)
        si = (si0, si1, si2, si3)
        r3 = (r3a, r3b)
        r4 = (r4a, r4b)
        s3 = (s3a, s3b)
        s4 = (s4a, s4b)
        w3 = (w3a, w3b)
        w4 = (w4a, w4b)
        sc = (sc0, sc1, sc2, sc3)

        @pl.loop(0, CHUNK)
        def _(i):
            for j in range(D // LANES):
                ones_v[i, pl.ds(j * LANES, LANES)] = jnp.ones((LANES,), jnp.float32)
                r4a[i, pl.ds(j * LANES, LANES)] = jnp.zeros((LANES,), jnp.float32)

        zr0 = sid * RPT
        for k in range(RPT // CHUNK):
            pltpu.sync_copy(r4a, cnt_sh.at[pl.ds(zr0 + k * CHUNK, CHUNK)])
        _t0 = (RPT // CHUNK) * CHUNK
        _tn = RPT - _t0
        pltpu.sync_copy(r4a.at[pl.ds(0, _tn)], cnt_sh.at[pl.ds(zr0 + _t0, _tn)])

        @pl.when(sid == NS - 1)
        def _():
            pltpu.sync_copy(r4a.at[pl.ds(0, TAILN)],
                            cnt_sh.at[pl.ds(TAIL0, TAILN)])

        plsc.subcore_barrier()

        def idx_load(c, s):
            eb = base + c * CHUNK
            pltpu.async_copy(src_hbm.at[pl.ds(eb, CHUNK)], isl[s], si[s])
            pltpu.async_copy(dst_hbm.at[pl.ds(eb, CHUNK)], idl[s], si[s])

        def idx_wait(c, s):
            eb = base + c * CHUNK
            pltpu.make_async_copy(src_hbm.at[pl.ds(eb, CHUNK)], isl[s], si[s]).wait()
            pltpu.make_async_copy(dst_hbm.at[pl.ds(eb, CHUNK)], idl[s], si[s]).wait()

        def fetch(c, b, s):
            pltpu.async_copy(x3_hbm.at[isl[s]], r3[b], s3[b])
            pltpu.async_copy(x4_hbm.at[idl[s]], r4[b], s4[b])

        def cnt_issue(s):
            pltpu.async_copy(ones_v, cnt_sh.at[isl[s]], sc[s], add=True)

        def cnt_wait(s):
            pltpu.make_async_copy(ones_v, cnt_sh.at[isl[s]], sc[s]).wait()

        def process(c, b, s, pf_i, pf_g):
            eb = base + c * CHUNK
            pltpu.make_async_copy(x3_hbm.at[isl[s]], r3[b], s3[b]).wait()
            pltpu.make_async_copy(x4_hbm.at[idl[s]], r4[b], s4[b]).wait()
            pltpu.async_copy(r3[b], g3_hbm.at[pl.ds(eb, CHUNK)], w3[b])
            pltpu.async_copy(r4[b], g4_hbm.at[pl.ds(eb, CHUNK)], w4[b])
            if pf_i:
                cnt_wait(s)
                idx_load(c + 4, s)
            if pf_g:
                s2 = (s + 2) % 4
                idx_wait(c + 2, s2)
                cnt_issue(s2)
                pltpu.make_async_copy(r3[b], g3_hbm.at[pl.ds(eb, CHUNK)], w3[b]).wait()
                pltpu.make_async_copy(r4[b], g4_hbm.at[pl.ds(eb, CHUNK)], w4[b]).wait()
                fetch(c + 2, b, s2)
            else:
                pltpu.make_async_copy(r3[b], g3_hbm.at[pl.ds(eb, CHUNK)], w3[b]).wait()
                pltpu.make_async_copy(r4[b], g4_hbm.at[pl.ds(eb, CHUNK)], w4[b]).wait()

        idx_load(0, 0)
        idx_load(1, 1)
        idx_load(2, 2)
        idx_load(3, 3)
        idx_wait(0, 0)
        cnt_issue(0)
        fetch(0, 0, 0)
        idx_wait(1, 1)
        cnt_issue(1)
        fetch(1, 1, 1)

        @pl.loop(0, NCHUNK - 6, step=4)
        def _(ci):
            process(ci + 0, 0, 0, True, True)
            process(ci + 1, 1, 1, True, True)
            process(ci + 2, 0, 2, True, True)
            process(ci + 3, 1, 3, True, True)

        process(NCHUNK - 6, 0, 0, True, True)
        process(NCHUNK - 5, 1, 1, True, True)
        process(NCHUNK - 4, 0, 2, False, True)
        process(NCHUNK - 3, 1, 3, False, True)
        process(NCHUNK - 2, 0, 0, False, False)
        process(NCHUNK - 1, 1, 1, False, False)

        cnt_wait(2)
        cnt_wait(3)
        cnt_wait(0)
        cnt_wait(1)

        plsc.subcore_barrier()

        r0 = sid * RPT
        pltpu.sync_copy(cnt_sh.at[pl.ds(r0, RPT)],
                        cnt_hbm.at[cid, pl.ds(r0, RPT)])

        @pl.when(sid == NS - 1)
        def _():
            pltpu.sync_copy(cnt_sh.at[pl.ds(TAIL0, TAILN)],
                            cnt_hbm.at[cid, pl.ds(TAIL0, TAILN)])

    return sc_gather34


@functools.cache
def _make_sc_msg_scatter():
    mesh = plsc.VectorSubcoreMesh(core_axis_name="c", subcore_axis_name="s")

    @functools.partial(
        pl.kernel,
        out_type=jax.ShapeDtypeStruct((NC, N, D), jnp.float32),
        mesh=mesh,
        scratch_types=[
            pltpu.VMEM((CHUNK,), jnp.int32),
            pltpu.VMEM((CHUNK,), jnp.int32),
            pltpu.VMEM((CHUNK,), jnp.int32),
            pltpu.VMEM((CHUNK,), jnp.int32),
            pltpu.VMEM((CHUNK,), jnp.int32),
            pltpu.VMEM((CHUNK,), jnp.int32),
            pltpu.VMEM((CHUNK,), jnp.int32),
            pltpu.VMEM((CHUNK,), jnp.int32),
            pltpu.VMEM((CHUNK, D), jnp.float32),
            pltpu.VMEM((CHUNK, D), jnp.float32),
            pltpu.VMEM((CHUNK, D), jnp.float32),
            pltpu.VMEM((CHUNK, D), jnp.float32),
            pltpu.VMEM_SHARED((N, D), jnp.float32),
            pltpu.SemaphoreType.DMA,
            pltpu.SemaphoreType.DMA,
            pltpu.SemaphoreType.DMA,
            pltpu.SemaphoreType.DMA,
            pltpu.SemaphoreType.DMA,
            pltpu.SemaphoreType.DMA,
            pltpu.SemaphoreType.DMA,
            pltpu.SemaphoreType.DMA,
        ],
    )
    def sc_msg_scatter(x2_hbm, sig_hbm, src_hbm, dst_hbm, sum_hbm,
                       is0, is1, is2, is3, id0, id1, id2, id3,
                       rows0, rows1, sig0, sig1, acc_sh,
                       si0, si1, si2, si3, sg0, sg1, ss0, ss1):
        cid = lax.axis_index("c")
        sid = lax.axis_index("s")
        wid = sid * NC + cid
        base = wid * EW
        isl = (is0, is1, is2, is3)
        idl = (id0, id1, id2, id3)
        si = (si0, si1, si2, si3)
        rows = (rows0, rows1)
        sigv = (sig0, sig1)
        sg = (sg0, sg1)
        ss = (ss0, ss1)

        @pl.loop(0, CHUNK)
        def _(i):
            for j in range(D // LANES):
                rows0[i, pl.ds(j * LANES, LANES)] = jnp.zeros((LANES,), jnp.float32)

        zr0 = sid * RPT
        for k in range(RPT // CHUNK):
            pltpu.sync_copy(rows0, acc_sh.at[pl.ds(zr0 + k * CHUNK, CHUNK)])
        _t0 = (RPT // CHUNK) * CHUNK
        _tn = RPT - _t0
        pltpu.sync_copy(rows0.at[pl.ds(0, _tn)], acc_sh.at[pl.ds(zr0 + _t0, _tn)])

        @pl.when(sid == NS - 1)
        def _():
            pltpu.sync_copy(rows0.at[pl.ds(0, TAILN)],
                            acc_sh.at[pl.ds(TAIL0, TAILN)])

        plsc.subcore_barrier()

        def idx_load(c, s):
            eb = base + c * CHUNK
            pltpu.async_copy(src_hbm.at[pl.ds(eb, CHUNK)], isl[s], si[s])
            pltpu.async_copy(dst_hbm.at[pl.ds(eb, CHUNK)], idl[s], si[s])

        def idx_wait(c, s):
            eb = base + c * CHUNK
            pltpu.make_async_copy(src_hbm.at[pl.ds(eb, CHUNK)], isl[s], si[s]).wait()
            pltpu.make_async_copy(dst_hbm.at[pl.ds(eb, CHUNK)], idl[s], si[s]).wait()

        def fetch(c, b, s):
            eb = base + c * CHUNK
            pltpu.async_copy(x2_hbm.at[idl[s]], rows[b], sg[b])
            pltpu.async_copy(sig_hbm.at[pl.ds(eb, CHUNK)], sigv[b], ss[b])

        def process(c, b, s, pf_i, pf_g):
            eb = base + c * CHUNK
            pltpu.make_async_copy(x2_hbm.at[idl[s]], rows[b], sg[b]).wait()
            pltpu.make_async_copy(sig_hbm.at[pl.ds(eb, CHUNK)], sigv[b], ss[b]).wait()

            @pl.loop(0, CHUNK, step=2)
            def _(i):
                for u in range(2):
                    for j in range(D // LANES):
                        sl = (i + u, pl.ds(j * LANES, LANES))
                        rows[b][sl] = rows[b][sl] * sigv[b][sl]

            pltpu.sync_copy(rows[b], acc_sh.at[isl[s]], add=True)
            if pf_i:
                idx_load(c + 4, s)
            if pf_g:
                s2 = (s + 2) % 4
                idx_wait(c + 2, s2)
                fetch(c + 2, b, s2)

        idx_load(0, 0)
        idx_load(1, 1)
        idx_load(2, 2)
        idx_load(3, 3)
        idx_wait(0, 0)
        fetch(0, 0, 0)
        idx_wait(1, 1)
        fetch(1, 1, 1)

        @pl.loop(0, NCHUNK - 6, step=4)
        def _(ci):
            process(ci + 0, 0, 0, True, True)
            process(ci + 1, 1, 1, True, True)
            process(ci + 2, 0, 2, True, True)
            process(ci + 3, 1, 3, True, True)

        process(NCHUNK - 6, 0, 0, True, True)
        process(NCHUNK - 5, 1, 1, True, True)
        process(NCHUNK - 4, 0, 2, False, True)
        process(NCHUNK - 3, 1, 3, False, True)
        process(NCHUNK - 2, 0, 0, False, False)
        process(NCHUNK - 1, 1, 1, False, False)

        plsc.subcore_barrier()

        r0 = sid * RPT
        pltpu.sync_copy(acc_sh.at[pl.ds(r0, RPT)],
                        sum_hbm.at[cid, pl.ds(r0, RPT)])

        @pl.when(sid == NS - 1)
        def _():
            pltpu.sync_copy(acc_sh.at[pl.ds(TAIL0, TAILN)],
                            sum_hbm.at[cid, pl.ds(TAIL0, TAILN)])

    return sc_msg_scatter


_NB = 1000
_EB = 4000


def _node_mm_body(x_ref, w1_ref, b1_ref, w2_ref, b2_ref, w3_ref, b3_ref,
                  w4_ref, b4_ref, o1_ref, o2_ref, o3_ref, o4_ref):
    xb = x_ref[...]
    o1_ref[...] = jnp.dot(xb, w1_ref[...], preferred_element_type=jnp.float32) + b1_ref[...]
    o2_ref[...] = jnp.dot(xb, w2_ref[...], preferred_element_type=jnp.float32) + b2_ref[...]
    o3_ref[...] = jnp.dot(xb, w3_ref[...], preferred_element_type=jnp.float32) + b3_ref[...]
    o4_ref[...] = jnp.dot(xb, w4_ref[...], preferred_element_type=jnp.float32) + b4_ref[...]


def _node_mm(x, w1t, b1, w2t, b2, w3t, b3, w4t, b4):
    row = pl.BlockSpec((_NB, D), lambda i: (i, 0))
    full = pl.BlockSpec((D, D), lambda i: (0, 0))
    bias = pl.BlockSpec((1, D), lambda i: (0, 0))
    o = jax.ShapeDtypeStruct((N, D), jnp.float32)
    return pl.pallas_call(
        _node_mm_body,
        grid=(N // _NB,),
        in_specs=[row, full, bias, full, bias, full, bias, full, bias],
        out_specs=[row, row, row, row],
        out_shape=[o, o, o, o],
    )(x, w1t, b1, w2t, b2, w3t, b3, w4t, b4)


def _edge1_body(w0_ref, we_ref, bwe_ref, sig_ref, w1_ref):
    w0b = w0_ref[...]
    sig_ref[...] = jax.nn.sigmoid(w0b)
    w1_ref[...] = jnp.dot(w0b, we_ref[...], preferred_element_type=jnp.float32) + bwe_ref[...]


def _edge1(w0, wet, bwe):
    row = pl.BlockSpec((_EB, D), lambda i: (i, 0))
    o = jax.ShapeDtypeStruct((E, D), jnp.float32)
    return pl.pallas_call(
        _edge1_body,
        grid=(E // _EB,),
        in_specs=[row, pl.BlockSpec((D, D), lambda i: (0, 0)),
                  pl.BlockSpec((1, D), lambda i: (0, 0))],
        out_specs=[row, row],
        out_shape=[o, o],
    )(w0, wet, bwe)


def _edge_stats_body(w1_ref, g3_ref, g4_ref, t_ref, s_ref, q_ref):
    i = pl.program_id(0)
    t = w1_ref[...] + g3_ref[...] + g4_ref[...]
    t_ref[...] = t
    ts = t.reshape(_EB // 8, 8, D)
    ps = jnp.sum(ts, axis=0)
    pq = jnp.sum(ts * ts, axis=0)

    @pl.when(i == 0)
    def _():
        s_ref[...] = jnp.zeros_like(s_ref)
        q_ref[...] = jnp.zeros_like(q_ref)

    s_ref[...] += ps
    q_ref[...] += pq


def _edge_stats(w1, g3, g4):
    row = pl.BlockSpec((_EB, D), lambda i: (i, 0))
    acc = pl.BlockSpec((8, D), lambda i: (0, 0))
    o = jax.ShapeDtypeStruct((8, D), jnp.float32)
    return pl.pallas_call(
        _edge_stats_body,
        grid=(E // _EB,),
        in_specs=[row, row, row],
        out_specs=[row, acc, acc],
        out_shape=[jax.ShapeDtypeStruct((E, D), jnp.float32), o, o],
    )(w1, g3, g4)


def _edge_fin_body(w0_ref, t_ref, s_ref, q_ref, ge_ref, be_ref, o_ref):
    s = jnp.sum(s_ref[...], axis=0, keepdims=True)
    q = jnp.sum(q_ref[...], axis=0, keepdims=True)
    m = s / E
    v = q / E - m * m
    r = lax.rsqrt(v + 1e-5)
    h = (t_ref[...] - m) * r * ge_ref[...] + be_ref[...]
    o_ref[...] = w0_ref[...] + h * jax.nn.sigmoid(h)


def _edge_fin(w0, t, ssum, ssq, ge, be):
    row = pl.BlockSpec((_EB, D), lambda i: (i, 0))
    acc = pl.BlockSpec((8, D), lambda i: (0, 0))
    bias = pl.BlockSpec((1, D), lambda i: (0, 0))
    return pl.pallas_call(
        _edge_fin_body,
        grid=(E // _EB,),
        in_specs=[row, row, acc, acc, bias, bias],
        out_specs=row,
        out_shape=jax.ShapeDtypeStruct((E, D), jnp.float32),
    )(w0, t, ssum, ssq, ge, be)


def _node_fin_body(x_ref, x1_ref, sp_ref, cp_ref, gv_ref, bv_ref, o_ref):
    s = sp_ref[0] + sp_ref[1]
    c = cp_ref[0, :, 0:1] + cp_ref[1, :, 0:1]
    agg = s / jnp.maximum(c, 1.0)
    h = x1_ref[...] + agg
    m = jnp.mean(h, axis=0, keepdims=True)
    v = jnp.mean((h - m) ** 2, axis=0, keepdims=True)
    hn = (h - m) * lax.rsqrt(v + 1e-5) * gv_ref[...] + bv_ref[...]
    o_ref[...] = x_ref[...] + hn * jax.nn.sigmoid(hn)


def _node_fin(x, x1, sum_p, cnt_p, gv, bv):
    return pl.pallas_call(
        _node_fin_body,
        grid=(1,),
        in_specs=[
            pl.BlockSpec((N, D), lambda i: (0, 0)),
            pl.BlockSpec((N, D), lambda i: (0, 0)),
            pl.BlockSpec((NC, N, D), lambda i: (0, 0, 0)),
            pl.BlockSpec((NC, N, D), lambda i: (0, 0, 0)),
            pl.BlockSpec((1, D), lambda i: (0, 0)),
            pl.BlockSpec((1, D), lambda i: (0, 0)),
        ],
        out_specs=pl.BlockSpec((N, D), lambda i: (0, 0)),
        out_shape=jax.ShapeDtypeStruct((N, D), jnp.float32),
    )(x, x1, sum_p, cnt_p, gv, bv)


def kernel(x, edge_index, edge_attr, W1, b1, W2, b2, W3, b3, W4, b4,
           We, bwe, g_v, beta_v, g_e, beta_e):
    src = edge_index[0]
    dst = edge_index[1]

    x1, x2, x3, x4 = _node_mm(
        x, W1.T, b1.reshape(1, D), W2.T, b2.reshape(1, D),
        W3.T, b3.reshape(1, D), W4.T, b4.reshape(1, D))

    g3, g4, cnt_p = _make_sc_gather34()(x3, x4, src, dst)

    sig, w1 = _edge1(edge_attr, We.T, bwe.reshape(1, D))

    sum_p = _make_sc_msg_scatter()(x2, sig, src, dst)

    t, ssum, ssq = _edge_stats(w1, g3, g4)

    x_out = _node_fin(x, x1, sum_p, cnt_p,
                      g_v.reshape(1, D), beta_v.reshape(1, D))
    w_out = _edge_fin(edge_attr, t, ssum, ssq,
                      g_e.reshape(1, D), beta_e.reshape(1, D))
    return (x_out, w_out)


# speedup vs baseline: 1.1792x; 1.0015x over previous
import functools

import jax
import jax.numpy as jnp
from jax import lax
from jax.experimental import pallas as pl
from jax.experimental.pallas import tpu as pltpu
from jax.experimental.pallas import tpu_sc as plsc

N = 10000
E = 320000
D = 128

NC = 2
NS = 16
NW = NC * NS
LANES = 16

CHUNK = 40
EW = E // NW
NCHUNK = EW // CHUNK
RPT = 624
TAIL0 = RPT * NS
TAILN = N - TAIL0


@functools.cache
def _make_sc_gather34():
    mesh = plsc.VectorSubcoreMesh(core_axis_name="c", subcore_axis_name="s")

    @functools.partial(
        pl.kernel,
        out_type=(
            jax.ShapeDtypeStruct((E, D), jnp.float32),
            jax.ShapeDtypeStruct((E, D), jnp.float32),
            jax.ShapeDtypeStruct((NC, N, D), jnp.float32),
        ),
        mesh=mesh,
        scratch_types=[
            pltpu.VMEM((CHUNK,), jnp.int32),
            pltpu.VMEM((CHUNK,), jnp.int32),
            pltpu.VMEM((CHUNK,), jnp.int32),
            pltpu.VMEM((CHUNK,), jnp.int32),
            pltpu.VMEM((CHUNK,), jnp.int32),
            pltpu.VMEM((CHUNK,), jnp.int32),
            pltpu.VMEM((CHUNK,), jnp.int32),
            pltpu.VMEM((CHUNK,), jnp.int32),
            pltpu.VMEM((CHUNK, D), jnp.float32),
            pltpu.VMEM((CHUNK, D), jnp.float32),
            pltpu.VMEM((CHUNK, D), jnp.float32),
            pltpu.VMEM((CHUNK, D), jnp.float32),
            pltpu.VMEM((CHUNK, D), jnp.float32),
            pltpu.SemaphoreType.DMA,
            pltpu.SemaphoreType.DMA,
            pltpu.SemaphoreType.DMA,
            pltpu.SemaphoreType.DMA,
            pltpu.SemaphoreType.DMA,
            pltpu.SemaphoreType.DMA,
            pltpu.SemaphoreType.DMA,
            pltpu.SemaphoreType.DMA,
            pltpu.SemaphoreType.DMA,
            pltpu.SemaphoreType.DMA,
            pltpu.SemaphoreType.DMA,
            pltpu.SemaphoreType.DMA,
            pltpu.SemaphoreType.DMA,
            pltpu.SemaphoreType.DMA,
            pltpu.SemaphoreType.DMA,
            pltpu.SemaphoreType.DMA,
            pltpu.VMEM_SHARED((N, D), jnp.float32),
        ],
    )
    def sc_gather34(x3_hbm, x4_hbm, src_hbm, dst_hbm, g3_hbm, g4_hbm, cnt_hbm,
                    is0, is1, is2, is3, id0, id1, id2, id3,
                    r3a, r3b, r4a, r4b, ones_v,
                    si0, si1, si2, si3, s3a, s3b, s4a, s4b, w3a, w3b, w4a, w4b,
                    sc0, sc1, sc2, sc3, cnt_sh):
        cid = lax.axis_index("c")
        sid = lax.axis_index("s")
        wid = sid * NC + cid
        base = wid * EW
        isl = (is0, is1, is2, is3)
        idl = (id0, id1, id2, id3)
        si = (si0, si1, si2, si3)
        r3 = (r3a, r3b)
        r4 = (r4a, r4b)
        s3 = (s3a, s3b)
        s4 = (s4a, s4b)
        w3 = (w3a, w3b)
        w4 = (w4a, w4b)
        sc = (sc0, sc1, sc2, sc3)

        @pl.loop(0, CHUNK)
        def _(i):
            for j in range(D // LANES):
                ones_v[i, pl.ds(j * LANES, LANES)] = jnp.ones((LANES,), jnp.float32)
                r4a[i, pl.ds(j * LANES, LANES)] = jnp.zeros((LANES,), jnp.float32)

        zr0 = sid * RPT
        for k in range(RPT // CHUNK):
            pltpu.sync_copy(r4a, cnt_sh.at[pl.ds(zr0 + k * CHUNK, CHUNK)])
        _t0 = (RPT // CHUNK) * CHUNK
        _tn = RPT - _t0
        pltpu.sync_copy(r4a.at[pl.ds(0, _tn)], cnt_sh.at[pl.ds(zr0 + _t0, _tn)])

        @pl.when(sid == NS - 1)
        def _():
            pltpu.sync_copy(r4a.at[pl.ds(0, TAILN)],
                            cnt_sh.at[pl.ds(TAIL0, TAILN)])

        plsc.subcore_barrier()

        def idx_load(c, s):
            eb = base + c * CHUNK
            pltpu.async_copy(src_hbm.at[pl.ds(eb, CHUNK)], isl[s], si[s])
            pltpu.async_copy(dst_hbm.at[pl.ds(eb, CHUNK)], idl[s], si[s])

        def idx_wait(c, s):
            eb = base + c * CHUNK
            pltpu.make_async_copy(src_hbm.at[pl.ds(eb, CHUNK)], isl[s], si[s]).wait()
            pltpu.make_async_copy(dst_hbm.at[pl.ds(eb, CHUNK)], idl[s], si[s]).wait()

        def fetch(c, b, s):
            pltpu.async_copy(x3_hbm.at[isl[s]], r3[b], s3[b])
            pltpu.async_copy(x4_hbm.at[idl[s]], r4[b], s4[b])

        def cnt_issue(s):
            pltpu.async_copy(ones_v, cnt_sh.at[isl[s]], sc[s], add=True)

        def cnt_wait(s):
            pltpu.make_async_copy(ones_v, cnt_sh.at[isl[s]], sc[s]).wait()

        def process(c, b, s, pf_i, pf_g):
            eb = base + c * CHUNK
            pltpu.make_async_copy(x3_hbm.at[isl[s]], r3[b], s3[b]).wait()
            pltpu.make_async_copy(x4_hbm.at[idl[s]], r4[b], s4[b]).wait()
            pltpu.async_copy(r3[b], g3_hbm.at[pl.ds(eb, CHUNK)], w3[b])
            pltpu.async_copy(r4[b], g4_hbm.at[pl.ds(eb, CHUNK)], w4[b])
            if pf_i:
                cnt_wait(s)
                idx_load(c + 4, s)
            if pf_g:
                s2 = (s + 2) % 4
                idx_wait(c + 2, s2)
                cnt_issue(s2)
                pltpu.make_async_copy(r3[b], g3_hbm.at[pl.ds(eb, CHUNK)], w3[b]).wait()
                pltpu.make_async_copy(r4[b], g4_hbm.at[pl.ds(eb, CHUNK)], w4[b]).wait()
                fetch(c + 2, b, s2)
            else:
                pltpu.make_async_copy(r3[b], g3_hbm.at[pl.ds(eb, CHUNK)], w3[b]).wait()
                pltpu.make_async_copy(r4[b], g4_hbm.at[pl.ds(eb, CHUNK)], w4[b]).wait()

        idx_load(0, 0)
        idx_load(1, 1)
        idx_load(2, 2)
        idx_load(3, 3)
        idx_wait(0, 0)
        cnt_issue(0)
        fetch(0, 0, 0)
        idx_wait(1, 1)
        cnt_issue(1)
        fetch(1, 1, 1)

        @pl.loop(0, NCHUNK - 6, step=4)
        def _(ci):
            process(ci + 0, 0, 0, True, True)
            process(ci + 1, 1, 1, True, True)
            process(ci + 2, 0, 2, True, True)
            process(ci + 3, 1, 3, True, True)

        process(NCHUNK - 6, 0, 0, True, True)
        process(NCHUNK - 5, 1, 1, True, True)
        process(NCHUNK - 4, 0, 2, False, True)
        process(NCHUNK - 3, 1, 3, False, True)
        process(NCHUNK - 2, 0, 0, False, False)
        process(NCHUNK - 1, 1, 1, False, False)

        cnt_wait(2)
        cnt_wait(3)
        cnt_wait(0)
        cnt_wait(1)

        plsc.subcore_barrier()

        r0 = sid * RPT
        pltpu.sync_copy(cnt_sh.at[pl.ds(r0, RPT)],
                        cnt_hbm.at[cid, pl.ds(r0, RPT)])

        @pl.when(sid == NS - 1)
        def _():
            pltpu.sync_copy(cnt_sh.at[pl.ds(TAIL0, TAILN)],
                            cnt_hbm.at[cid, pl.ds(TAIL0, TAILN)])

    return sc_gather34


@functools.cache
def _make_sc_msg_scatter():
    mesh = plsc.VectorSubcoreMesh(core_axis_name="c", subcore_axis_name="s")

    @functools.partial(
        pl.kernel,
        out_type=jax.ShapeDtypeStruct((NC, N, D), jnp.float32),
        mesh=mesh,
        scratch_types=[
            pltpu.VMEM((CHUNK,), jnp.int32),
            pltpu.VMEM((CHUNK,), jnp.int32),
            pltpu.VMEM((CHUNK,), jnp.int32),
            pltpu.VMEM((CHUNK,), jnp.int32),
            pltpu.VMEM((CHUNK,), jnp.int32),
            pltpu.VMEM((CHUNK,), jnp.int32),
            pltpu.VMEM((CHUNK,), jnp.int32),
            pltpu.VMEM((CHUNK,), jnp.int32),
            pltpu.VMEM((CHUNK, D), jnp.float32),
            pltpu.VMEM((CHUNK, D), jnp.float32),
            pltpu.VMEM((CHUNK, D), jnp.float32),
            pltpu.VMEM((CHUNK, D), jnp.float32),
            pltpu.VMEM_SHARED((N, D), jnp.float32),
            pltpu.SemaphoreType.DMA,
            pltpu.SemaphoreType.DMA,
            pltpu.SemaphoreType.DMA,
            pltpu.SemaphoreType.DMA,
            pltpu.SemaphoreType.DMA,
            pltpu.SemaphoreType.DMA,
            pltpu.SemaphoreType.DMA,
            pltpu.SemaphoreType.DMA,
        ],
    )
    def sc_msg_scatter(x2_hbm, sig_hbm, src_hbm, dst_hbm, sum_hbm,
                       is0, is1, is2, is3, id0, id1, id2, id3,
                       rows0, rows1, sig0, sig1, acc_sh,
                       si0, si1, si2, si3, sg0, sg1, ss0, ss1):
        cid = lax.axis_index("c")
        sid = lax.axis_index("s")
        wid = sid * NC + cid
        base = wid * EW
        isl = (is0, is1, is2, is3)
        idl = (id0, id1, id2, id3)
        si = (si0, si1, si2, si3)
        rows = (rows0, rows1)
        sigv = (sig0, sig1)
        sg = (sg0, sg1)
        ss = (ss0, ss1)

        @pl.loop(0, CHUNK)
        def _(i):
            for j in range(D // LANES):
                rows0[i, pl.ds(j * LANES, LANES)] = jnp.zeros((LANES,), jnp.float32)

        zr0 = sid * RPT
        for k in range(RPT // CHUNK):
            pltpu.sync_copy(rows0, acc_sh.at[pl.ds(zr0 + k * CHUNK, CHUNK)])
        _t0 = (RPT // CHUNK) * CHUNK
        _tn = RPT - _t0
        pltpu.sync_copy(rows0.at[pl.ds(0, _tn)], acc_sh.at[pl.ds(zr0 + _t0, _tn)])

        @pl.when(sid == NS - 1)
        def _():
            pltpu.sync_copy(rows0.at[pl.ds(0, TAILN)],
                            acc_sh.at[pl.ds(TAIL0, TAILN)])

        plsc.subcore_barrier()

        def idx_load(c, s):
            eb = base + c * CHUNK
            pltpu.async_copy(src_hbm.at[pl.ds(eb, CHUNK)], isl[s], si[s])
            pltpu.async_copy(dst_hbm.at[pl.ds(eb, CHUNK)], idl[s], si[s])

        def idx_wait(c, s):
            eb = base + c * CHUNK
            pltpu.make_async_copy(src_hbm.at[pl.ds(eb, CHUNK)], isl[s], si[s]).wait()
            pltpu.make_async_copy(dst_hbm.at[pl.ds(eb, CHUNK)], idl[s], si[s]).wait()

        def fetch(c, b, s):
            eb = base + c * CHUNK
            pltpu.async_copy(x2_hbm.at[idl[s]], rows[b], sg[b])
            pltpu.async_copy(sig_hbm.at[pl.ds(eb, CHUNK)], sigv[b], ss[b])

        def process(c, b, s, pf_i, pf_g):
            eb = base + c * CHUNK
            pltpu.make_async_copy(x2_hbm.at[idl[s]], rows[b], sg[b]).wait()
            pltpu.make_async_copy(sig_hbm.at[pl.ds(eb, CHUNK)], sigv[b], ss[b]).wait()

            @plsc.parallel_loop(0, CHUNK, step=1, unroll=4)
            def _(i):
                for j in range(D // LANES):
                    sl = (i, pl.ds(j * LANES, LANES))
                    rows[b][sl] = rows[b][sl] * sigv[b][sl]

            pltpu.sync_copy(rows[b], acc_sh.at[isl[s]], add=True)
            if pf_i:
                idx_load(c + 4, s)
            if pf_g:
                s2 = (s + 2) % 4
                idx_wait(c + 2, s2)
                fetch(c + 2, b, s2)

        idx_load(0, 0)
        idx_load(1, 1)
        idx_load(2, 2)
        idx_load(3, 3)
        idx_wait(0, 0)
        fetch(0, 0, 0)
        idx_wait(1, 1)
        fetch(1, 1, 1)

        @pl.loop(0, NCHUNK - 6, step=4)
        def _(ci):
            process(ci + 0, 0, 0, True, True)
            process(ci + 1, 1, 1, True, True)
            process(ci + 2, 0, 2, True, True)
            process(ci + 3, 1, 3, True, True)

        process(NCHUNK - 6, 0, 0, True, True)
        process(NCHUNK - 5, 1, 1, True, True)
        process(NCHUNK - 4, 0, 2, False, True)
        process(NCHUNK - 3, 1, 3, False, True)
        process(NCHUNK - 2, 0, 0, False, False)
        process(NCHUNK - 1, 1, 1, False, False)

        plsc.subcore_barrier()

        r0 = sid * RPT
        pltpu.sync_copy(acc_sh.at[pl.ds(r0, RPT)],
                        sum_hbm.at[cid, pl.ds(r0, RPT)])

        @pl.when(sid == NS - 1)
        def _():
            pltpu.sync_copy(acc_sh.at[pl.ds(TAIL0, TAILN)],
                            sum_hbm.at[cid, pl.ds(TAIL0, TAILN)])

    return sc_msg_scatter


_NB = 1000
_EB = 4000


def _node_mm_body(x_ref, w1_ref, b1_ref, w2_ref, b2_ref, w3_ref, b3_ref,
                  w4_ref, b4_ref, o1_ref, o2_ref, o3_ref, o4_ref):
    xb = x_ref[...]
    o1_ref[...] = jnp.dot(xb, w1_ref[...], preferred_element_type=jnp.float32) + b1_ref[...]
    o2_ref[...] = jnp.dot(xb, w2_ref[...], preferred_element_type=jnp.float32) + b2_ref[...]
    o3_ref[...] = jnp.dot(xb, w3_ref[...], preferred_element_type=jnp.float32) + b3_ref[...]
    o4_ref[...] = jnp.dot(xb, w4_ref[...], preferred_element_type=jnp.float32) + b4_ref[...]


def _node_mm(x, w1t, b1, w2t, b2, w3t, b3, w4t, b4):
    row = pl.BlockSpec((_NB, D), lambda i: (i, 0))
    full = pl.BlockSpec((D, D), lambda i: (0, 0))
    bias = pl.BlockSpec((1, D), lambda i: (0, 0))
    o = jax.ShapeDtypeStruct((N, D), jnp.float32)
    return pl.pallas_call(
        _node_mm_body,
        grid=(N // _NB,),
        in_specs=[row, full, bias, full, bias, full, bias, full, bias],
        out_specs=[row, row, row, row],
        out_shape=[o, o, o, o],
    )(x, w1t, b1, w2t, b2, w3t, b3, w4t, b4)


def _edge1_body(w0_ref, we_ref, bwe_ref, sig_ref, w1_ref):
    w0b = w0_ref[...]
    sig_ref[...] = jax.nn.sigmoid(w0b)
    w1_ref[...] = jnp.dot(w0b, we_ref[...], preferred_element_type=jnp.float32) + bwe_ref[...]


def _edge1(w0, wet, bwe):
    row = pl.BlockSpec((_EB, D), lambda i: (i, 0))
    o = jax.ShapeDtypeStruct((E, D), jnp.float32)
    return pl.pallas_call(
        _edge1_body,
        grid=(E // _EB,),
        in_specs=[row, pl.BlockSpec((D, D), lambda i: (0, 0)),
                  pl.BlockSpec((1, D), lambda i: (0, 0))],
        out_specs=[row, row],
        out_shape=[o, o],
    )(w0, wet, bwe)


def _edge_stats_body(w1_ref, g3_ref, g4_ref, t_ref, s_ref, q_ref):
    i = pl.program_id(0)
    t = w1_ref[...] + g3_ref[...] + g4_ref[...]
    t_ref[...] = t
    ts = t.reshape(_EB // 8, 8, D)
    ps = jnp.sum(ts, axis=0)
    pq = jnp.sum(ts * ts, axis=0)

    @pl.when(i == 0)
    def _():
        s_ref[...] = jnp.zeros_like(s_ref)
        q_ref[...] = jnp.zeros_like(q_ref)

    s_ref[...] += ps
    q_ref[...] += pq


def _edge_stats(w1, g3, g4):
    row = pl.BlockSpec((_EB, D), lambda i: (i, 0))
    acc = pl.BlockSpec((8, D), lambda i: (0, 0))
    o = jax.ShapeDtypeStruct((8, D), jnp.float32)
    return pl.pallas_call(
        _edge_stats_body,
        grid=(E // _EB,),
        in_specs=[row, row, row],
        out_specs=[row, acc, acc],
        out_shape=[jax.ShapeDtypeStruct((E, D), jnp.float32), o, o],
    )(w1, g3, g4)


def _edge_fin_body(w0_ref, t_ref, s_ref, q_ref, ge_ref, be_ref, o_ref):
    s = jnp.sum(s_ref[...], axis=0, keepdims=True)
    q = jnp.sum(q_ref[...], axis=0, keepdims=True)
    m = s / E
    v = q / E - m * m
    r = lax.rsqrt(v + 1e-5)
    h = (t_ref[...] - m) * r * ge_ref[...] + be_ref[...]
    o_ref[...] = w0_ref[...] + h * jax.nn.sigmoid(h)


def _edge_fin(w0, t, ssum, ssq, ge, be):
    row = pl.BlockSpec((_EB, D), lambda i: (i, 0))
    acc = pl.BlockSpec((8, D), lambda i: (0, 0))
    bias = pl.BlockSpec((1, D), lambda i: (0, 0))
    return pl.pallas_call(
        _edge_fin_body,
        grid=(E // _EB,),
        in_specs=[row, row, acc, acc, bias, bias],
        out_specs=row,
        out_shape=jax.ShapeDtypeStruct((E, D), jnp.float32),
    )(w0, t, ssum, ssq, ge, be)


def _node_fin_body(x_ref, x1_ref, sp_ref, cp_ref, gv_ref, bv_ref, o_ref):
    s = sp_ref[0] + sp_ref[1]
    c = cp_ref[0, :, 0:1] + cp_ref[1, :, 0:1]
    agg = s / jnp.maximum(c, 1.0)
    h = x1_ref[...] + agg
    m = jnp.mean(h, axis=0, keepdims=True)
    v = jnp.mean((h - m) ** 2, axis=0, keepdims=True)
    hn = (h - m) * lax.rsqrt(v + 1e-5) * gv_ref[...] + bv_ref[...]
    o_ref[...] = x_ref[...] + hn * jax.nn.sigmoid(hn)


def _node_fin(x, x1, sum_p, cnt_p, gv, bv):
    return pl.pallas_call(
        _node_fin_body,
        grid=(1,),
        in_specs=[
            pl.BlockSpec((N, D), lambda i: (0, 0)),
            pl.BlockSpec((N, D), lambda i: (0, 0)),
            pl.BlockSpec((NC, N, D), lambda i: (0, 0, 0)),
            pl.BlockSpec((NC, N, D), lambda i: (0, 0, 0)),
            pl.BlockSpec((1, D), lambda i: (0, 0)),
            pl.BlockSpec((1, D), lambda i: (0, 0)),
        ],
        out_specs=pl.BlockSpec((N, D), lambda i: (0, 0)),
        out_shape=jax.ShapeDtypeStruct((N, D), jnp.float32),
    )(x, x1, sum_p, cnt_p, gv, bv)


def kernel(x, edge_index, edge_attr, W1, b1, W2, b2, W3, b3, W4, b4,
           We, bwe, g_v, beta_v, g_e, beta_e):
    src = edge_index[0]
    dst = edge_index[1]

    x1, x2, x3, x4 = _node_mm(
        x, W1.T, b1.reshape(1, D), W2.T, b2.reshape(1, D),
        W3.T, b3.reshape(1, D), W4.T, b4.reshape(1, D))

    g3, g4, cnt_p = _make_sc_gather34()(x3, x4, src, dst)

    sig, w1 = _edge1(edge_attr, We.T, bwe.reshape(1, D))

    sum_p = _make_sc_msg_scatter()(x2, sig, src, dst)

    t, ssum, ssq = _edge_stats(w1, g3, g4)

    x_out = _node_fin(x, x1, sum_p, cnt_p,
                      g_v.reshape(1, D), beta_v.reshape(1, D))
    w_out = _edge_fin(edge_attr, t, ssum, ssq,
                      g_e.reshape(1, D), beta_e.reshape(1, D))
    return (x_out, w_out)
